# Initial kernel scaffold; baseline (speedup 1.0000x reference)
import functools

import jax
import jax.numpy as jnp
from jax import lax
from jax.experimental import pallas as pl
from jax.experimental.pallas import tpu as pltpu
from jax.experimental.pallas import tpu_sc as plsc

N = 10000
NPAD = 10240
D_IN = 128
HID = 64
OUT = 64
NHEAD = 8
DH = OUT // NHEAD
FF = 2048
NMP = 3

SP_ROWS = NMP * NPAD
NTILES = 32
CHUNK = 128
KCH = 8
STEP = KCH * CHUNK
ROWS_PER_TILE = SP_ROWS // 16


_B1 = 2000


def _pre_body(x_ref, w_ref, o_ref):
    o_ref[0] = jnp.dot(x_ref[...], w_ref[0], preferred_element_type=jnp.float32)


def _pre_project(x, wl):
    return pl.pallas_call(
        _pre_body,
        grid=(NMP, N // _B1),
        in_specs=[
            pl.BlockSpec((_B1, D_IN), lambda m, i: (i, 0)),
            pl.BlockSpec((1, D_IN, HID), lambda m, i: (m, 0, 0)),
        ],
        out_specs=pl.BlockSpec((1, _B1, HID), lambda m, i: (m, i, 0)),
        out_shape=jax.ShapeDtypeStruct((NMP, N, HID), jnp.float32),
    )(x, wl)


EDGES_PER_TILE = 30720
EDGES_PADDED = EDGES_PER_TILE * NTILES
NSTEPS = EDGES_PER_TILE // STEP


@functools.partial(
    pl.kernel,
    out_type=jax.ShapeDtypeStruct((2, SP_ROWS, HID), jnp.float32),
    mesh=plsc.VectorSubcoreMesh(core_axis_name="c", subcore_axis_name="s"),
    scratch_types=[
        pltpu.VMEM((KCH, CHUNK), jnp.int32),
        pltpu.VMEM((KCH, CHUNK), jnp.int32),
        pltpu.VMEM((STEP, HID), jnp.float32),
        pltpu.VMEM((CHUNK, HID), jnp.float32),
        pltpu.VMEM_SHARED((SP_ROWS, HID), jnp.float32),
        pltpu.SemaphoreType.DMA,
    ],
)
def _sc_scatter(ytab, srcs, dsts, out, src_v, dst_v, rows_v, zeros_v, acc, sem):
    cid = lax.axis_index("c")
    sid = lax.axis_index("s")
    wid = cid * 16 + sid

    def _zrow(r, carry):
        for c4 in range(HID // 16):
            zeros_v[r, pl.ds(c4 * 16, 16)] = jnp.zeros((16,), jnp.float32)
        return carry

    lax.fori_loop(0, CHUNK, _zrow, 0)

    def _zacc(j, carry):
        pltpu.sync_copy(zeros_v, acc.at[pl.ds(sid * ROWS_PER_TILE + j * CHUNK, CHUNK)])
        return carry

    lax.fori_loop(0, ROWS_PER_TILE // CHUNK, _zacc, 0)
    plsc.subcore_barrier()

    base_row = wid * (EDGES_PER_TILE // CHUNK)

    def _step(g, carry):
        row0 = base_row + g * KCH
        pltpu.sync_copy(srcs.at[pl.ds(row0, KCH)], src_v)
        pltpu.sync_copy(dsts.at[pl.ds(row0, KCH)], dst_v)
        copies = []
        for j in range(KCH):
            copies.append(
                pltpu.async_copy(
                    ytab.at[src_v.at[j]],
                    rows_v.at[pl.ds(j * CHUNK, CHUNK)],
                    sem,
                )
            )
        for cp in copies:
            cp.wait()
        for j in range(KCH):
            pltpu.sync_copy(
                rows_v.at[pl.ds(j * CHUNK, CHUNK)],
                acc.at[dst_v.at[j]],
                add=True,
            )
        return carry

    lax.fori_loop(0, NSTEPS, _step, 0)
    plsc.subcore_barrier()

    pltpu.sync_copy(
        acc.at[pl.ds(sid * ROWS_PER_TILE, ROWS_PER_TILE)],
        out.at[cid, pl.ds(sid * ROWS_PER_TILE, ROWS_PER_TILE)],
    )


_B2 = 400


def _ln(v, s, b):
    mu = jnp.mean(v, axis=-1, keepdims=True)
    var = jnp.mean((v - mu) ** 2, axis=-1, keepdims=True)
    return (v - mu) * jax.lax.rsqrt(var + 1e-5) * s + b


def _post_body(x_ref, agg_ref, w01_ref, b0_ref, outw_ref, outb_ref,
               wq_ref, bq_ref, lns_ref, lnb_ref,
               ff1_ref, ff1b_ref, ff2_ref, ff2b_ref,
               h1_ref, h1b_ref, h2_ref, h2b_ref, o_ref):
    x = x_ref[...]
    zs = []
    for m in range(NMP):
        aggm = agg_ref[0, m] + agg_ref[1, m]
        hm = aggm + jnp.dot(x, w01_ref[m], preferred_element_type=jnp.float32)
        hm = jnp.maximum(hm + b0_ref[m], 0.0)
        em = jnp.dot(hm, outw_ref[m], preferred_element_type=jnp.float32)
        zs.append((em + outb_ref[m]) * (1.0 / NMP))

    gi = lax.broadcasted_iota(jnp.int32, (OUT, NHEAD), 0) // DH
    gh = lax.broadcasted_iota(jnp.int32, (OUT, NHEAD), 1)
    G = (gi == gh).astype(jnp.float32)
    scale = 1.0 / (DH ** 0.5)

    for l in range(2):
        q = [jnp.dot(z, wq_ref[l, 0], preferred_element_type=jnp.float32) + bq_ref[l, 0] for z in zs]
        k = [jnp.dot(z, wq_ref[l, 1], preferred_element_type=jnp.float32) + bq_ref[l, 1] for z in zs]
        v = [jnp.dot(z, wq_ref[l, 2], preferred_element_type=jnp.float32) + bq_ref[l, 2] for z in zs]
        new_zs = []
        for i in range(NMP):
            s_ij = [jnp.dot(q[i] * k[j], G, preferred_element_type=jnp.float32) * scale
                    for j in range(NMP)]
            smax = jnp.maximum(jnp.maximum(s_ij[0], s_ij[1]), s_ij[2])
            e_ij = [jnp.exp(s - smax) for s in s_ij]
            den = e_ij[0] + e_ij[1] + e_ij[2]
            o_acc = jnp.zeros_like(zs[i])
            for j in range(NMP):
                w = jnp.dot(e_ij[j] / den, G.T, preferred_element_type=jnp.float32)
                o_acc = o_acc + w * v[j]
            attn = jnp.dot(o_acc, wq_ref[l, 3], preferred_element_type=jnp.float32) + bq_ref[l, 3]
            new_zs.append(_ln(zs[i] + attn, lns_ref[l, 0], lnb_ref[l, 0]))
        zcat = jnp.concatenate(new_zs, axis=0)
        f = jnp.maximum(jnp.dot(zcat, ff1_ref[l], preferred_element_type=jnp.float32) + ff1b_ref[l], 0.0)
        f = jnp.dot(f, ff2_ref[l], preferred_element_type=jnp.float32) + ff2b_ref[l]
        zs = [_ln(new_zs[i] + f[i * _B2:(i + 1) * _B2], lns_ref[l, 1], lnb_ref[l, 1])
              for i in range(NMP)]

    pooled = (zs[0] + zs[1] + zs[2]) * (1.0 / NMP)
    h = jnp.maximum(jnp.dot(pooled, h1_ref[...], preferred_element_type=jnp.float32) + h1b_ref[...], 0.0)
    o_ref[...] = jnp.dot(h, h2_ref[...], preferred_element_type=jnp.float32) + h2b_ref[...]


def _full(shape):
    nd = len(shape)
    return pl.BlockSpec(shape, lambda i, _n=nd: (0,) * _n)


def _post(x, agg, weights):
    in_specs = [
        pl.BlockSpec((_B2, D_IN), lambda i: (i, 0)),
        pl.BlockSpec((2, NMP, _B2, HID), lambda i: (0, 0, i, 0)),
    ] + [_full(w.shape) for w in weights]
    return pl.pallas_call(
        _post_body,
        grid=(N // _B2,),
        in_specs=in_specs,
        out_specs=pl.BlockSpec((_B2, 1), lambda i: (i, 0)),
        out_shape=jax.ShapeDtypeStruct((N, 1), jnp.float32),
    )(x, agg, *weights)


def _stack_weights(params):
    mps = [params['mp%d' % m] for m in range(NMP)]
    w01 = jnp.stack([p['w0_W'] + p['w1_W'] for p in mps])
    b0 = jnp.stack([(p['w0_b'] + p['w1_b'] + p['wl_b'])[None, :] for p in mps])
    outw = jnp.stack([p['out_W'] for p in mps])
    outb = jnp.stack([p['out_b'][None, :] for p in mps])
    encs = [params['enc%d' % l] for l in range(2)]
    wq = jnp.stack([jnp.stack([p['W' + nm] for nm in ('q', 'k', 'v', 'o')]) for p in encs])
    bq = jnp.stack([jnp.stack([p['b' + nm][None, :] for nm in ('q', 'k', 'v', 'o')]) for p in encs])
    lns = jnp.stack([jnp.stack([p['ln1_s'][None, :], p['ln2_s'][None, :]]) for p in encs])
    lnb = jnp.stack([jnp.stack([p['ln1_b'][None, :], p['ln2_b'][None, :]]) for p in encs])
    ff1 = jnp.stack([p['ff1_W'] for p in encs])
    ff1b = jnp.stack([p['ff1_b'][None, :] for p in encs])
    ff2 = jnp.stack([p['ff2_W'] for p in encs])
    ff2b = jnp.stack([p['ff2_b'][None, :] for p in encs])
    h1 = params['head1_W']
    h1b = params['head1_b'][None, :]
    h2 = params['head2_W']
    h2b = params['head2_b'][None, :]
    return (w01, b0, outw, outb, wq, bq, lns, lnb,
            ff1, ff1b, ff2, ff2b, h1, h1b, h2, h2b)


def _edge_lists(e0, e1, e2):
    srcs = jnp.concatenate([e0[1], e1[1] + N, e2[1] + 2 * N]).astype(jnp.int32)
    dsts = jnp.concatenate(
        [e0[0], e1[0] + NPAD, e2[0] + 2 * NPAD]).astype(jnp.int32)
    pad = EDGES_PADDED - srcs.shape[0]
    srcs = jnp.concatenate([srcs, jnp.zeros((pad,), jnp.int32)])
    dsts = jnp.concatenate([dsts, jnp.full((pad,), N, jnp.int32)])
    return srcs.reshape(-1, CHUNK), dsts.reshape(-1, CHUNK)


def kernel(x, edge_index_rel0, edge_index_rel1, edge_index_rel2, params):
    wl = jnp.stack([params['mp%d' % m]['wl_W'] for m in range(NMP)])
    ytab = _pre_project(x, wl).reshape(NMP * N, HID)
    srcs, dsts = _edge_lists(edge_index_rel0, edge_index_rel1, edge_index_rel2)
    parts = _sc_scatter(ytab, srcs, dsts)
    agg = parts.reshape(2, NMP, NPAD, HID)
    out = _post(x, agg, _stack_weights(params))
    return out[:, 0]


# speedup vs baseline: 4.4371x; 4.4371x over previous
import functools

import jax
import jax.numpy as jnp
from jax import lax
from jax.experimental import pallas as pl
from jax.experimental.pallas import tpu as pltpu
from jax.experimental.pallas import tpu_sc as plsc

N = 10000
NPAD = 10240
D_IN = 128
HID = 64
OUT = 64
NHEAD = 8
DH = OUT // NHEAD
FF = 2048
NMP = 3

NTILES = 32
CHUNK = 128
KCH = 8
STEP = KCH * CHUNK
ROWS_PER_TILE = NPAD // 16


_B1 = 2000


def _pre_body(x_ref, w_ref, o_ref):
    o_ref[0] = jnp.dot(x_ref[...], w_ref[0], preferred_element_type=jnp.float32)


def _pre_project(x, wl):
    return pl.pallas_call(
        _pre_body,
        grid=(NMP, N // _B1),
        in_specs=[
            pl.BlockSpec((_B1, D_IN), lambda m, i: (i, 0)),
            pl.BlockSpec((1, D_IN, HID), lambda m, i: (m, 0, 0)),
        ],
        out_specs=pl.BlockSpec((1, _B1, HID), lambda m, i: (m, i, 0)),
        out_shape=jax.ShapeDtypeStruct((NMP, N, HID), jnp.float32),
    )(x, wl)


EDGES_PER_TILE = 10240
EDGES_PADDED = EDGES_PER_TILE * NTILES
NSTEPS = EDGES_PER_TILE // STEP
REL_ROWS = EDGES_PADDED // CHUNK


def _sc_scatter_body(ytab, srcs, dsts, out, src_v, dst_v, rows_v, zeros_v, acc, sem):
    cid = lax.axis_index("c")
    sid = lax.axis_index("s")
    wid = cid * 16 + sid

    def _zrow(r, carry):
        for c4 in range(HID // 16):
            zeros_v[r, pl.ds(c4 * 16, 16)] = jnp.zeros((16,), jnp.float32)
        return carry

    lax.fori_loop(0, CHUNK, _zrow, 0)

    for m in range(NMP):
        def _zacc(j, carry):
            pltpu.sync_copy(
                zeros_v,
                acc.at[pl.ds(sid * ROWS_PER_TILE + j * CHUNK, CHUNK)])
            return carry

        lax.fori_loop(0, ROWS_PER_TILE // CHUNK, _zacc, 0)
        plsc.subcore_barrier()

        base_row = m * REL_ROWS + wid * (EDGES_PER_TILE // CHUNK)

        def _step(g, carry):
            row0 = base_row + g * KCH
            pltpu.sync_copy(srcs.at[pl.ds(row0, KCH)], src_v)
            pltpu.sync_copy(dsts.at[pl.ds(row0, KCH)], dst_v)
            copies = []
            for j in range(KCH):
                copies.append(
                    pltpu.async_copy(
                        ytab.at[src_v.at[j]],
                        rows_v.at[pl.ds(j * CHUNK, CHUNK)],
                        sem,
                    )
                )
            for cp in copies:
                cp.wait()
            for j in range(KCH):
                pltpu.sync_copy(
                    rows_v.at[pl.ds(j * CHUNK, CHUNK)],
                    acc.at[dst_v.at[j]],
                    add=True,
                )
            return carry

        lax.fori_loop(0, NSTEPS, _step, 0)
        plsc.subcore_barrier()

        pltpu.sync_copy(
            acc.at[pl.ds(sid * ROWS_PER_TILE, ROWS_PER_TILE)],
            out.at[cid, m, pl.ds(sid * ROWS_PER_TILE, ROWS_PER_TILE)],
        )


_SC_SCATTER_CACHE = []


def _sc_scatter(ytab, srcs, dsts):
    if not _SC_SCATTER_CACHE:
        fn = functools.partial(
            pl.kernel,
            out_type=jax.ShapeDtypeStruct((2, NMP, NPAD, HID), jnp.float32),
            mesh=plsc.VectorSubcoreMesh(
                core_axis_name="c", subcore_axis_name="s"),
            compiler_params=pltpu.CompilerParams(use_tc_tiling_on_sc=False),
            scratch_types=[
                pltpu.VMEM((KCH, CHUNK), jnp.int32),
                pltpu.VMEM((KCH, CHUNK), jnp.int32),
                pltpu.VMEM((STEP, HID), jnp.float32),
                pltpu.VMEM((CHUNK, HID), jnp.float32),
                pltpu.VMEM_SHARED((NPAD, HID), jnp.float32),
                pltpu.SemaphoreType.DMA,
            ],
        )(_sc_scatter_body)
        _SC_SCATTER_CACHE.append(fn)
    return _SC_SCATTER_CACHE[0](ytab, srcs, dsts)


_B2 = 400


def _ln(v, s, b):
    mu = jnp.mean(v, axis=-1, keepdims=True)
    var = jnp.mean((v - mu) ** 2, axis=-1, keepdims=True)
    return (v - mu) * jax.lax.rsqrt(var + 1e-5) * s + b


def _bf(a):
    return a.astype(jnp.bfloat16)


def _bdot(a, b):
    return jnp.dot(_bf(a), _bf(b), preferred_element_type=jnp.float32)


def _post_body(x_ref, agg_ref, w01_ref, b0_ref, outw_ref, outb_ref,
               wq_ref, bq_ref, lns_ref, lnb_ref,
               ff1_ref, ff1b_ref, ff2_ref, ff2b_ref,
               h1_ref, h1b_ref, h2_ref, h2b_ref, o_ref):
    x = x_ref[...]
    zs = []
    for m in range(NMP):
        aggm = agg_ref[0, m] + agg_ref[1, m]
        hm = aggm + _bdot(x, w01_ref[m])
        hm = jnp.maximum(hm + b0_ref[m], 0.0)
        em = _bdot(hm, outw_ref[m])
        zs.append((em + outb_ref[m]) * (1.0 / NMP))

    gi = lax.broadcasted_iota(jnp.int32, (OUT, NHEAD), 0) // DH
    gh = lax.broadcasted_iota(jnp.int32, (OUT, NHEAD), 1)
    G = (gi == gh).astype(jnp.float32)
    scale = 1.0 / (DH ** 0.5)

    for l in range(2):
        q = [_bdot(z, wq_ref[l, 0]) + bq_ref[l, 0] for z in zs]
        k = [_bdot(z, wq_ref[l, 1]) + bq_ref[l, 1] for z in zs]
        v = [_bdot(z, wq_ref[l, 2]) + bq_ref[l, 2] for z in zs]
        qb = [_bf(a).astype(jnp.float32) for a in q]
        kb = [_bf(a).astype(jnp.float32) for a in k]
        vb = [_bf(a).astype(jnp.float32) for a in v]
        new_zs = []
        for i in range(NMP):
            s_ij = [jnp.dot(qb[i] * kb[j], G, preferred_element_type=jnp.float32) * scale
                    for j in range(NMP)]
            smax = jnp.maximum(jnp.maximum(s_ij[0], s_ij[1]), s_ij[2])
            e_ij = [jnp.exp(s - smax) for s in s_ij]
            den = e_ij[0] + e_ij[1] + e_ij[2]
            o_acc = jnp.zeros_like(zs[i])
            for j in range(NMP):
                w = jnp.dot(_bf(e_ij[j] / den).astype(jnp.float32), G.T,
                            preferred_element_type=jnp.float32)
                o_acc = o_acc + w * vb[j]
            attn = _bdot(o_acc, wq_ref[l, 3]) + bq_ref[l, 3]
            new_zs.append(_ln(zs[i] + attn, lns_ref[l, 0], lnb_ref[l, 0]))
        zcat = jnp.concatenate(new_zs, axis=0)
        f = jnp.maximum(_bdot(zcat, ff1_ref[l]) + ff1b_ref[l], 0.0)
        f = _bdot(f, ff2_ref[l]) + ff2b_ref[l]
        zs = [_ln(new_zs[i] + f[i * _B2:(i + 1) * _B2], lns_ref[l, 1], lnb_ref[l, 1])
              for i in range(NMP)]

    pooled = (zs[0] + zs[1] + zs[2]) * (1.0 / NMP)
    h = jnp.maximum(_bdot(pooled, h1_ref[...]) + h1b_ref[...], 0.0)
    o_ref[...] = _bdot(h, h2_ref[...]) + h2b_ref[...]


def _full(shape):
    nd = len(shape)
    return pl.BlockSpec(shape, lambda i, _n=nd: (0,) * _n)


def _post(x, agg, weights):
    in_specs = [
        pl.BlockSpec((_B2, D_IN), lambda i: (i, 0)),
        pl.BlockSpec((2, NMP, _B2, HID), lambda i: (0, 0, i, 0)),
    ] + [_full(w.shape) for w in weights]
    return pl.pallas_call(
        _post_body,
        grid=(N // _B2,),
        in_specs=in_specs,
        out_specs=pl.BlockSpec((_B2, 1), lambda i: (i, 0)),
        out_shape=jax.ShapeDtypeStruct((N, 1), jnp.float32),
    )(x, agg, *weights)


def _stack_weights(params):
    mps = [params['mp%d' % m] for m in range(NMP)]
    w01 = jnp.stack([p['w0_W'] + p['w1_W'] for p in mps])
    b0 = jnp.stack([(p['w0_b'] + p['w1_b'] + p['wl_b'])[None, :] for p in mps])
    outw = jnp.stack([p['out_W'] for p in mps])
    outb = jnp.stack([p['out_b'][None, :] for p in mps])
    encs = [params['enc%d' % l] for l in range(2)]
    wq = jnp.stack([jnp.stack([p['W' + nm] for nm in ('q', 'k', 'v', 'o')]) for p in encs])
    bq = jnp.stack([jnp.stack([p['b' + nm][None, :] for nm in ('q', 'k', 'v', 'o')]) for p in encs])
    lns = jnp.stack([jnp.stack([p['ln1_s'][None, :], p['ln2_s'][None, :]]) for p in encs])
    lnb = jnp.stack([jnp.stack([p['ln1_b'][None, :], p['ln2_b'][None, :]]) for p in encs])
    ff1 = jnp.stack([p['ff1_W'] for p in encs])
    ff1b = jnp.stack([p['ff1_b'][None, :] for p in encs])
    ff2 = jnp.stack([p['ff2_W'] for p in encs])
    ff2b = jnp.stack([p['ff2_b'][None, :] for p in encs])
    h1 = params['head1_W']
    h1b = params['head1_b'][None, :]
    h2 = params['head2_W']
    h2b = params['head2_b'][None, :]
    return (w01, b0, outw, outb, wq, bq, lns, lnb,
            ff1, ff1b, ff2, ff2b, h1, h1b, h2, h2b)


def _edge_lists(e0, e1, e2):
    es = [e0, e1, e2]
    pad = EDGES_PADDED - es[0].shape[1]
    srcs = jnp.concatenate(
        [jnp.concatenate([es[m][1].astype(jnp.int32) + m * N,
                          jnp.zeros((pad,), jnp.int32)])
         for m in range(NMP)])
    dsts = jnp.concatenate(
        [jnp.concatenate([es[m][0].astype(jnp.int32),
                          jnp.full((pad,), N, jnp.int32)])
         for m in range(NMP)])
    return srcs.reshape(-1, CHUNK), dsts.reshape(-1, CHUNK)


def kernel(x, edge_index_rel0, edge_index_rel1, edge_index_rel2, params):
    wl = jnp.stack([params['mp%d' % m]['wl_W'] for m in range(NMP)])
    ytab = _pre_project(x, wl).reshape(NMP * N, HID)
    srcs, dsts = _edge_lists(edge_index_rel0, edge_index_rel1, edge_index_rel2)
    agg = _sc_scatter(ytab, srcs, dsts)
    out = _post(x, agg, _stack_weights(params))
    return out[:, 0]
